# baseline (speedup 1.0000x reference)
import functools

import jax
import jax.numpy as jnp
from jax import lax
from jax.experimental import pallas as pl
from jax.experimental.pallas import tpu as pltpu
from jax.experimental.pallas import tpu_sc as plsc

N_LOC = 1000000
N_USER = 100000
D_MODEL = 64
BATCH = 4096
HIST = 200

NC = 2
NS = 16
NW = NC * NS
NBT = BATCH // 128
BTS = 2
SU_ROWS = 128 * BTS
NSU = HIST * NBT // BTS
SU_PER_W = NSU // NW
NBQ = NBT // BTS
TPAD = 129

RP_C = 384
RP_UNITS = N_LOC // RP_C
RP_TAIL_I0 = RP_UNITS * RP_C
RP_T = 82
RP_W = RP_C * D_MODEL


def _sc_repack_body(tT_hbm, tail_hbm, tP_hbm, buf_v, tbuf_v, sems):
    wid = lax.axis_index("s") * NC + lax.axis_index("c")
    rsem = (sems[0], sems[1])
    wsem = (sems[2], sems[3])
    iota16 = lax.iota(jnp.int32, 16)
    _diag = [(iota16 + k) % 16 for k in range(16)]
    _diag64 = [iota16 * 64 + d for d in _diag]

    def unit_of(t):
        return jnp.minimum(wid + NW * t, RP_UNITS - 1)

    def read_start(u, slot):
        i0 = pl.multiple_of(u * RP_C, 128)
        pltpu.async_copy(
            tT_hbm.at[pl.ds(0, D_MODEL), pl.ds(i0, RP_C)],
            buf_v.at[slot],
            rsem[slot],
        )

    def read_wait(slot):
        pltpu.make_async_copy(
            tT_hbm.at[pl.ds(0, D_MODEL), pl.ds(0, RP_C)],
            buf_v.at[slot],
            rsem[slot],
        ).wait()

    def write_start(u, slot):
        pltpu.async_copy(
            tbuf_v.at[slot], tP_hbm.at[pl.ds(u * RP_W, RP_W)], wsem[slot]
        )

    def write_wait(slot):
        pltpu.make_async_copy(
            tbuf_v.at[slot], tP_hbm.at[pl.ds(0, RP_W)], wsem[slot]
        ).wait()

    def transpose(slot):
        slot_vec = jnp.full((16,), slot, jnp.int32)

        def bbody(bi, carry):
            ivec = bi * 16 + iota16
            for j0 in range(0, D_MODEL, 16):
                vals = [
                    plsc.load_gather(buf_v, (slot_vec, j0 + _diag[k], ivec))
                    for k in range(16)
                ]
                base = bi * 1024 + j0
                for k in range(16):
                    plsc.store_scatter(
                        tbuf_v, (slot_vec, _diag64[k] + base), vals[k]
                    )
            return carry

        lax.fori_loop(0, RP_C // 16, bbody, 0)

    def unit_body(t, slot, has_prev, has_next):
        if has_next:
            if has_prev:
                write_wait(1 - slot)
            read_start(unit_of(t + 1), 1 - slot)
        read_wait(slot)
        transpose(slot)
        write_start(unit_of(t), slot)

    read_start(unit_of(0), 0)
    unit_body(0, 0, False, True)
    unit_body(1, 1, True, True)

    def pair(g, carry):
        t = 2 + 2 * g
        unit_body(t, 0, True, True)
        unit_body(t + 1, 1, True, True)
        return carry

    lax.fori_loop(0, (RP_T - 4) // 2, pair, 0)

    unit_body(RP_T - 2, 0, True, True)
    unit_body(RP_T - 1, 1, False, False)
    write_wait(0)
    write_wait(1)

    @pl.when(wid == 0)
    def _():
        pltpu.sync_copy(tail_hbm, tbuf_v.at[0, pl.ds(0, 4096)])
        pltpu.sync_copy(
            tbuf_v.at[0, pl.ds(0, 4096)],
            tP_hbm.at[pl.ds(RP_TAIL_I0 * D_MODEL, 4096)],
        )


@functools.cache
def _sc_repack():
    return pl.kernel(
        _sc_repack_body,
        out_type=jax.ShapeDtypeStruct((N_LOC * D_MODEL,), jnp.float32),
        mesh=plsc.VectorSubcoreMesh(
            core_axis_name="c", subcore_axis_name="s", num_cores=NC, num_subcores=NS
        ),
        compiler_params=pltpu.CompilerParams(
            needs_layout_passes=False, use_tc_tiling_on_sc=True
        ),
        scratch_types=[
            pltpu.VMEM((2, D_MODEL, RP_C), jnp.float32),
            pltpu.VMEM((2, RP_W), jnp.float32),
            [pltpu.SemaphoreType.DMA] * 4,
        ],
    )


def _sc_gather_body(idx_hbm, table_hbm, out_hbm, idx_v, rows_v, trans_v, sems):
    wid = lax.axis_index("s") * NC + lax.axis_index("c")
    su_base = wid * SU_PER_W
    gsem = (sems[0], sems[1])
    ssem = (sems[2], sems[3])
    isem = (sems[4], sems[5])
    iota16 = lax.iota(jnp.int32, 16)

    def idx_start(su, slot):
        h = su // NBQ
        bq = su % NBQ
        pltpu.async_copy(
            idx_hbm.at[h // 8, pl.ds(BTS * bq, BTS)], idx_v.at[slot], isem[slot]
        )

    def idx_wait(slot):
        pltpu.make_async_copy(
            idx_hbm.at[0, pl.ds(0, BTS)], idx_v.at[slot], isem[slot]
        ).wait()

    def gather_start(su, slot):
        hr = lax.rem(su // NBQ, 8)
        for j in range(BTS):
            pltpu.async_copy(
                table_hbm.at[idx_v.at[slot, j, hr]],
                rows_v.at[slot, pl.ds(j * 128, 128)],
                gsem[slot],
            )

    def gather_wait(slot):
        pltpu.make_async_copy(
            table_hbm.at[pl.ds(0, SU_ROWS)], rows_v.at[slot], gsem[slot]
        ).wait()

    def scatter_start(su, slot):
        h = su // NBQ
        bq = su % NBQ
        pltpu.async_copy(
            trans_v.at[slot, :, :, :, pl.ds(0, 128)],
            out_hbm.at[h, :, pl.ds(BTS * bq, BTS)],
            ssem[slot],
        )

    def scatter_wait(slot):
        pltpu.make_async_copy(
            trans_v.at[slot, :, :, :, pl.ds(0, 128)],
            out_hbm.at[0, :, pl.ds(0, BTS)],
            ssem[slot],
        ).wait()

    def fixup(su, slot):
        hr = lax.rem(su // NBQ, 8)
        slot_vec = jnp.full((16,), slot, jnp.int32)
        zeros_f = jnp.zeros((16,), jnp.float32)

        def group(g, carry):
            j = g // 8
            l = g - j * 8
            iv = idx_v[slot, j, hr, pl.ds(l * 16, 16)]
            nzero = plsc.all_reduce_population_count(iv == 0)

            @pl.when(nzero[0] > 0)
            def _():
                pos = g * 16 + iota16
                msk = iv == 0

                def col_body(col, c2):
                    colv = jnp.full((16,), 0, jnp.int32) + col
                    plsc.store_scatter(
                        rows_v, (slot_vec, pos, colv), zeros_f, mask=msk
                    )
                    return c2

                lax.fori_loop(0, D_MODEL, col_body, 0)

            return carry

        lax.fori_loop(0, SU_ROWS // 16, group, 0)

    _jt = [(j0 * 16 + iota16) // 8 for j0 in range(D_MODEL // 16)]
    _jr = [(j0 * 16 + iota16) % 8 for j0 in range(D_MODEL // 16)]

    def transpose(slot):
        slot_vec = jnp.full((16,), slot, jnp.int32)

        def tbody(b0, carry):
            vals = []
            for db in range(4):
                b = b0 * 4 + db
                for j0 in range(D_MODEL // 16):
                    vals.append(rows_v[slot, b, pl.ds(j0 * 16, 16)])
            k = 0
            for db in range(4):
                b = b0 * 4 + db
                btp_vec = jnp.full((16,), 0, jnp.int32) + (b // 128)
                bl_vec = jnp.full((16,), 0, jnp.int32) + (b % 128)
                for j0 in range(D_MODEL // 16):
                    plsc.store_scatter(
                        trans_v,
                        (slot_vec, _jt[j0], btp_vec, _jr[j0], bl_vec),
                        vals[k],
                    )
                    k += 1
            return carry

        lax.fori_loop(0, SU_ROWS // 4, tbody, 0)

    def unit_body(su, slot, has_prev, has_next, load_next):
        if has_next:
            if has_prev:
                scatter_wait(1 - slot)
            idx_wait(1 - slot)
            gather_start(su + 1, 1 - slot)
        gather_wait(slot)
        fixup(su, slot)
        transpose(slot)
        if load_next:
            idx_start(su + 2, slot)
        scatter_start(su, slot)

    idx_start(su_base + 0, 0)
    idx_wait(0)
    gather_start(su_base + 0, 0)
    idx_start(su_base + 1, 1)

    unit_body(su_base + 0, 0, False, True, True)
    unit_body(su_base + 1, 1, True, True, True)

    def pair(g, carry):
        su = su_base + 2 + 2 * g
        unit_body(su, 0, True, True, True)
        unit_body(su + 1, 1, True, True, True)
        return carry

    lax.fori_loop(0, (SU_PER_W - 4) // 2, pair, 0)

    unit_body(su_base + SU_PER_W - 2, 0, True, True, False)
    unit_body(su_base + SU_PER_W - 1, 1, False, False, False)

    scatter_wait(0)
    scatter_wait(1)


@functools.cache
def _sc_gather():
    return pl.kernel(
        _sc_gather_body,
        out_type=jax.ShapeDtypeStruct(
            (HIST, D_MODEL // 8, NBT, 8, 128), jnp.float32
        ),
        mesh=plsc.VectorSubcoreMesh(
            core_axis_name="c", subcore_axis_name="s", num_cores=NC, num_subcores=NS
        ),
        compiler_params=pltpu.CompilerParams(
            needs_layout_passes=False, use_tc_tiling_on_sc=False
        ),
        scratch_types=[
            pltpu.VMEM((2, BTS, 8, 128), jnp.int32),
            pltpu.VMEM((2, SU_ROWS, D_MODEL), jnp.float32),
            pltpu.VMEM((2, D_MODEL // 8, BTS, 8, TPAD), jnp.float32),
            [pltpu.SemaphoreType.DMA] * 6,
        ],
    )


_U_ROWS = 8


def _tc_copy_body(u_ref, t_ref, uo_ref, to_ref):
    i = pl.program_id(0)
    col = lax.broadcasted_iota(jnp.int32, (_U_ROWS, N_USER), 1)
    uo_ref[...] = jnp.where(col == 0, 0.0, u_ref[...])

    @pl.when(i == 0)
    def _():
        to_ref[...] = t_ref[...]


def _tc_copy(user_t, time_table):
    return pl.pallas_call(
        _tc_copy_body,
        grid=(D_MODEL // _U_ROWS,),
        in_specs=[
            pl.BlockSpec((_U_ROWS, N_USER), lambda i: (i, 0)),
            pl.BlockSpec((24, D_MODEL), lambda i: (0, 0)),
        ],
        out_specs=[
            pl.BlockSpec((_U_ROWS, N_USER), lambda i: (i, 0)),
            pl.BlockSpec((24, D_MODEL), lambda i: (0, 0)),
        ],
        out_shape=[
            jax.ShapeDtypeStruct((D_MODEL, N_USER), jnp.float32),
            jax.ShapeDtypeStruct((24, D_MODEL), jnp.float32),
        ],
    )(user_t, time_table)


def kernel(location_x, loc_table, user_table, time_table):
    idx_phys = location_x.T.reshape(HIST // 8, 8, NBT, 128).transpose(0, 2, 1, 3)
    tail = loc_table[RP_TAIL_I0:].reshape(-1)
    table_rm = _sc_repack()(loc_table.T, tail).reshape(N_LOC, D_MODEL)
    out5 = _sc_gather()(idx_phys, table_rm)
    loc_embedded = out5.transpose(2, 4, 0, 1, 3).reshape(BATCH, HIST, D_MODEL)
    user_t, timeslot_embedded = _tc_copy(user_table.T, time_table)
    return (loc_embedded, timeslot_embedded, user_t.T)


# speedup vs baseline: 1.6638x; 1.6638x over previous
import functools

import jax
import jax.numpy as jnp
from jax import lax
from jax.experimental import pallas as pl
from jax.experimental.pallas import tpu as pltpu
from jax.experimental.pallas import tpu_sc as plsc

N_LOC = 1000000
N_USER = 100000
D_MODEL = 64
BATCH = 4096
HIST = 200

NC = 2
NS = 16
NW = NC * NS
NBT = BATCH // 128
BTS = 2
SU_ROWS = 128 * BTS
NSU = HIST * NBT // BTS
SU_PER_W = NSU // NW
NBQ = NBT // BTS
TPAD = 129

RP_C = 384
RP_UNITS = N_LOC // RP_C
RP_TAIL_I0 = RP_UNITS * RP_C
RP_T = 82
RP_W = RP_C * D_MODEL


def _sc_repack_body(tT_hbm, tail_hbm, tP_hbm, buf_v, tbuf_v, sems):
    wid = lax.axis_index("s") * NC + lax.axis_index("c")
    rsem = (sems[0], sems[1])
    wsem = (sems[2], sems[3])
    iota16 = lax.iota(jnp.int32, 16)
    _diag = [(iota16 + k) % 16 for k in range(16)]
    _diag64 = [iota16 * 64 + d for d in _diag]

    def unit_of(t):
        return jnp.minimum(wid + NW * t, RP_UNITS - 1)

    def read_start(u, slot):
        i0 = pl.multiple_of(u * RP_C, 128)
        pltpu.async_copy(
            tT_hbm.at[pl.ds(0, D_MODEL), pl.ds(i0, RP_C)],
            buf_v.at[slot],
            rsem[slot],
        )

    def read_wait(slot):
        pltpu.make_async_copy(
            tT_hbm.at[pl.ds(0, D_MODEL), pl.ds(0, RP_C)],
            buf_v.at[slot],
            rsem[slot],
        ).wait()

    def write_start(u, slot):
        pltpu.async_copy(
            tbuf_v.at[slot], tP_hbm.at[pl.ds(u * RP_W, RP_W)], wsem[slot]
        )

    def write_wait(slot):
        pltpu.make_async_copy(
            tbuf_v.at[slot], tP_hbm.at[pl.ds(0, RP_W)], wsem[slot]
        ).wait()

    def transpose(slot):
        slot_vec = jnp.full((16,), slot, jnp.int32)

        def bbody(bi, carry):
            ivec = bi * 16 + iota16
            for j0 in range(0, D_MODEL, 16):
                vals = [
                    plsc.load_gather(buf_v, (slot_vec, j0 + _diag[k], ivec))
                    for k in range(16)
                ]
                base = bi * 1024 + j0
                for k in range(16):
                    plsc.store_scatter(
                        tbuf_v, (slot_vec, _diag64[k] + base), vals[k]
                    )
            return carry

        lax.fori_loop(0, RP_C // 16, bbody, 0)

    def unit_body(t, slot, has_prev, has_next):
        if has_next:
            if has_prev:
                write_wait(1 - slot)
            read_start(unit_of(t + 1), 1 - slot)
        read_wait(slot)
        transpose(slot)
        write_start(unit_of(t), slot)

    read_start(unit_of(0), 0)
    unit_body(0, 0, False, True)
    unit_body(1, 1, True, True)

    def pair(g, carry):
        t = 2 + 2 * g
        unit_body(t, 0, True, True)
        unit_body(t + 1, 1, True, True)
        return carry

    lax.fori_loop(0, (RP_T - 4) // 2, pair, 0)

    unit_body(RP_T - 2, 0, True, True)
    unit_body(RP_T - 1, 1, False, False)
    write_wait(0)
    write_wait(1)

    @pl.when(wid == 0)
    def _():
        pltpu.sync_copy(tail_hbm, tbuf_v.at[0, pl.ds(0, 4096)])
        pltpu.sync_copy(
            tbuf_v.at[0, pl.ds(0, 4096)],
            tP_hbm.at[pl.ds(RP_TAIL_I0 * D_MODEL, 4096)],
        )


@functools.cache
def _sc_repack():
    return pl.kernel(
        _sc_repack_body,
        out_type=jax.ShapeDtypeStruct((N_LOC * D_MODEL,), jnp.float32),
        mesh=plsc.VectorSubcoreMesh(
            core_axis_name="c", subcore_axis_name="s", num_cores=NC, num_subcores=NS
        ),
        compiler_params=pltpu.CompilerParams(
            needs_layout_passes=False, use_tc_tiling_on_sc=True
        ),
        scratch_types=[
            pltpu.VMEM((2, D_MODEL, RP_C), jnp.float32),
            pltpu.VMEM((2, RP_W), jnp.float32),
            [pltpu.SemaphoreType.DMA] * 4,
        ],
    )


def _sc_gather_body(idx_hbm, table_hbm, out_hbm, idx_v, rows_v, trans_v, sems):
    wid = lax.axis_index("s") * NC + lax.axis_index("c")
    su_base = wid * SU_PER_W
    gsem = (sems[0], sems[1], sems[2])
    ssem = (sems[3], sems[4], sems[5])
    isem = (sems[6], sems[7], sems[8])
    iota16 = lax.iota(jnp.int32, 16)

    def idx_start(su, slot):
        h = su // NBQ
        bq = su % NBQ
        pltpu.async_copy(
            idx_hbm.at[h // 8, pl.ds(BTS * bq, BTS)], idx_v.at[slot], isem[slot]
        )

    def idx_wait(slot):
        pltpu.make_async_copy(
            idx_hbm.at[0, pl.ds(0, BTS)], idx_v.at[slot], isem[slot]
        ).wait()

    def gather_start(su, slot):
        hr = lax.rem(su // NBQ, 8)
        for j in range(BTS):
            pltpu.async_copy(
                table_hbm.at[idx_v.at[slot, j, hr]],
                rows_v.at[slot, pl.ds(j * 128, 128)],
                gsem[slot],
            )

    def gather_wait(slot):
        pltpu.make_async_copy(
            table_hbm.at[pl.ds(0, SU_ROWS)], rows_v.at[slot], gsem[slot]
        ).wait()

    def scatter_start(su, slot):
        h = su // NBQ
        bq = su % NBQ
        pltpu.async_copy(
            trans_v.at[slot, :, :, :, pl.ds(0, 128)],
            out_hbm.at[h, :, pl.ds(BTS * bq, BTS)],
            ssem[slot],
        )

    def scatter_wait(slot):
        pltpu.make_async_copy(
            trans_v.at[slot, :, :, :, pl.ds(0, 128)],
            out_hbm.at[0, :, pl.ds(0, BTS)],
            ssem[slot],
        ).wait()

    def fixup(su, slot):
        hr = lax.rem(su // NBQ, 8)
        slot_vec = jnp.full((16,), slot, jnp.int32)
        zeros_f = jnp.zeros((16,), jnp.float32)

        def group(g, carry):
            j = g // 8
            l = g - j * 8
            iv = idx_v[slot, j, hr, pl.ds(l * 16, 16)]
            nzero = plsc.all_reduce_population_count(iv == 0)

            @pl.when(nzero[0] > 0)
            def _():
                pos = g * 16 + iota16
                msk = iv == 0

                def col_body(col, c2):
                    colv = jnp.full((16,), 0, jnp.int32) + col
                    plsc.store_scatter(
                        rows_v, (slot_vec, pos, colv), zeros_f, mask=msk
                    )
                    return c2

                lax.fori_loop(0, D_MODEL, col_body, 0)

            return carry

        lax.fori_loop(0, SU_ROWS // 16, group, 0)

    _jt = [(j0 * 16 + iota16) // 8 for j0 in range(D_MODEL // 16)]
    _jr = [(j0 * 16 + iota16) % 8 for j0 in range(D_MODEL // 16)]

    def transpose(slot):
        slot_vec = jnp.full((16,), slot, jnp.int32)

        def tbody(b0, carry):
            vals = []
            for db in range(4):
                b = b0 * 4 + db
                for j0 in range(D_MODEL // 16):
                    vals.append(rows_v[slot, b, pl.ds(j0 * 16, 16)])
            k = 0
            for db in range(4):
                b = b0 * 4 + db
                btp_vec = jnp.full((16,), 0, jnp.int32) + (b // 128)
                bl_vec = jnp.full((16,), 0, jnp.int32) + (b % 128)
                for j0 in range(D_MODEL // 16):
                    plsc.store_scatter(
                        trans_v,
                        (slot_vec, _jt[j0], btp_vec, _jr[j0], bl_vec),
                        vals[k],
                    )
                    k += 1
            return carry

        lax.fori_loop(0, SU_ROWS // 4, tbody, 0)

    def unit_body(su, slot, has_prev2, has_next, load_next):
        ns = (slot + 1) % 3
        if has_next:
            if has_prev2:
                scatter_wait(ns)
            idx_wait(ns)
            gather_start(su + 1, ns)
        gather_wait(slot)
        fixup(su, slot)
        transpose(slot)
        if load_next:
            idx_start(su + 3, slot)
        scatter_start(su, slot)

    idx_start(su_base + 0, 0)
    idx_wait(0)
    gather_start(su_base + 0, 0)
    idx_start(su_base + 1, 1)
    idx_start(su_base + 2, 2)

    unit_body(su_base + 0, 0, False, True, True)
    unit_body(su_base + 1, 1, False, True, True)
    unit_body(su_base + 2, 2, True, True, True)

    def triple(g, carry):
        su = su_base + 3 + 3 * g
        unit_body(su, 0, True, True, True)
        unit_body(su + 1, 1, True, True, True)
        unit_body(su + 2, 2, True, True, True)
        return carry

    lax.fori_loop(0, (SU_PER_W - 7) // 3, triple, 0)

    unit_body(su_base + SU_PER_W - 4, 0, True, True, True)
    unit_body(su_base + SU_PER_W - 3, 1, True, True, False)
    unit_body(su_base + SU_PER_W - 2, 2, True, True, False)
    unit_body(su_base + SU_PER_W - 1, 0, False, False, False)

    scatter_wait(1)
    scatter_wait(2)
    scatter_wait(0)


@functools.cache
def _sc_gather():
    return pl.kernel(
        _sc_gather_body,
        out_type=jax.ShapeDtypeStruct(
            (HIST, D_MODEL // 8, NBT, 8, 128), jnp.float32
        ),
        mesh=plsc.VectorSubcoreMesh(
            core_axis_name="c", subcore_axis_name="s", num_cores=NC, num_subcores=NS
        ),
        compiler_params=pltpu.CompilerParams(
            needs_layout_passes=False, use_tc_tiling_on_sc=False
        ),
        scratch_types=[
            pltpu.VMEM((3, BTS, 8, 128), jnp.int32),
            pltpu.VMEM((3, SU_ROWS, D_MODEL), jnp.float32),
            pltpu.VMEM((3, D_MODEL // 8, BTS, 8, TPAD), jnp.float32),
            [pltpu.SemaphoreType.DMA] * 9,
        ],
    )


_U_ROWS = 8


def _tc_copy_body(u_ref, t_ref, uo_ref, to_ref):
    i = pl.program_id(0)
    col = lax.broadcasted_iota(jnp.int32, (_U_ROWS, N_USER), 1)
    uo_ref[...] = jnp.where(col == 0, 0.0, u_ref[...])

    @pl.when(i == 0)
    def _():
        to_ref[...] = t_ref[...]


def _tc_copy(user_t, time_table):
    return pl.pallas_call(
        _tc_copy_body,
        grid=(D_MODEL // _U_ROWS,),
        in_specs=[
            pl.BlockSpec((_U_ROWS, N_USER), lambda i: (i, 0)),
            pl.BlockSpec((24, D_MODEL), lambda i: (0, 0)),
        ],
        out_specs=[
            pl.BlockSpec((_U_ROWS, N_USER), lambda i: (i, 0)),
            pl.BlockSpec((24, D_MODEL), lambda i: (0, 0)),
        ],
        out_shape=[
            jax.ShapeDtypeStruct((D_MODEL, N_USER), jnp.float32),
            jax.ShapeDtypeStruct((24, D_MODEL), jnp.float32),
        ],
    )(user_t, time_table)


def kernel(location_x, loc_table, user_table, time_table):
    idx_phys = location_x.T.reshape(HIST // 8, 8, NBT, 128).transpose(0, 2, 1, 3)
    tail = loc_table[RP_TAIL_I0:].reshape(-1)
    table_rm = _sc_repack()(loc_table.T, tail).reshape(N_LOC, D_MODEL)
    out5 = _sc_gather()(idx_phys, table_rm)
    loc_embedded = out5.transpose(2, 4, 0, 1, 3).reshape(BATCH, HIST, D_MODEL)
    user_t, timeslot_embedded = _tc_copy(user_table.T, time_table)
    return (loc_embedded, timeslot_embedded, user_t.T)
